# baseline (speedup 1.0000x reference)
import functools

import jax
import jax.numpy as jnp
from jax import lax
from jax.experimental import pallas as pl
from jax.experimental.pallas import tpu as pltpu
from jax.experimental.pallas import tpu_sc as plsc

B, H, D = 16384, 200, 64
N = B * H
NC, NS = 2, 16
NW = NC * NS
ROWS_PER_W = N // NW
IDX_MINOR = 128
CHUNK = 256
STREAMS = CHUNK // IDX_MINOR
N_CHUNKS = ROWS_PER_W // CHUNK
IDX_ROWS_W = ROWS_PER_W // IDX_MINOR
NBUF = 5
G = N_CHUNKS // NBUF

_mesh = plsc.VectorSubcoreMesh(core_axis_name="c", subcore_axis_name="s")


@functools.partial(
    pl.kernel,
    mesh=_mesh,
    out_type=jax.ShapeDtypeStruct((N, D), jnp.float32),
    compiler_params=pltpu.CompilerParams(use_tc_tiling_on_sc=False),
    scratch_types=(
        [pltpu.VMEM((STREAMS, IDX_MINOR), jnp.int32) for _ in range(NBUF)]
        + [pltpu.VMEM((CHUNK, D), jnp.float32) for _ in range(NBUF)]
        + [pltpu.SemaphoreType.DMA for _ in range(3 * NBUF)]
    ),
)
def _gather(idx_hbm, table_hbm, out_hbm, *bufs):
    idx_v = bufs[:NBUF]
    rows_v = bufs[NBUF:2 * NBUF]
    gsem = bufs[2 * NBUF:3 * NBUF]
    osem = bufs[3 * NBUF:4 * NBUF]
    isem = bufs[4 * NBUF:5 * NBUF]

    wid = lax.axis_index("s") * NC + lax.axis_index("c")
    idx_row0 = wid * IDX_ROWS_W
    out_row0 = wid * ROWS_PER_W

    def idx_load(b, c):
        return pltpu.make_async_copy(
            idx_hbm.at[pl.ds(idx_row0 + c * STREAMS, STREAMS)],
            idx_v[b], isem[b])

    def g_copy(b, j):
        return pltpu.make_async_copy(
            table_hbm.at[idx_v[b].at[j]],
            rows_v[b].at[pl.ds(j * IDX_MINOR, IDX_MINOR)], gsem[b])

    def o_copy(b, c):
        return pltpu.make_async_copy(
            rows_v[b], out_hbm.at[pl.ds(out_row0 + c * CHUNK, CHUNK)],
            osem[b])

    for m in range(NBUF):
        idx_load(m, m).start()
    for m in range(NBUF - 1):
        idx_load(m, m).wait()
        for j in range(STREAMS):
            g_copy(m, j).start()

    def step(g, carry):
        for b in range(NBUF):
            c = g * NBUF + b
            fb = (b - 1) % NBUF

            if b == 0:
                @pl.when(g > 0)
                def _():
                    o_copy(fb, c - 1).wait()
            else:
                o_copy(fb, c - 1).wait()

            def fire_ahead(c=c, fb=fb):
                idx_load(fb, c + NBUF - 1).wait()
                for j in range(STREAMS):
                    g_copy(fb, j).start()

            if b == 0:
                fire_ahead()
            else:
                pl.when(g < G - 1)(fire_ahead)

            for j in range(STREAMS):
                g_copy(b, j).wait()
            o_copy(b, c).start()

            @pl.when(g < G - 1)
            def _():
                idx_load(b, c + NBUF).start()
        return carry

    lax.fori_loop(0, G, step, 0)
    o_copy((N_CHUNKS - 1) % NBUF, N_CHUNKS - 1).wait()


def kernel(x, emb):
    idx = x.reshape(N // IDX_MINOR, IDX_MINOR).astype(jnp.int32)
    out = _gather(idx, emb)
    return out.reshape(B, H, D)


# speedup vs baseline: 1.0041x; 1.0041x over previous
import functools

import jax
import jax.numpy as jnp
from jax import lax
from jax.experimental import pallas as pl
from jax.experimental.pallas import tpu as pltpu
from jax.experimental.pallas import tpu_sc as plsc

B, H, D = 16384, 200, 64
NC, NS = 2, 16
NW = NC * NS
B_PER_W = B // NW
SB = 4
N_CHUNKS = B_PER_W // SB
NBUF = 2
G = N_CHUNKS // NBUF
SPLITS = ((0, 128), (128, 72))

_mesh = plsc.VectorSubcoreMesh(core_axis_name="c", subcore_axis_name="s")


@functools.partial(
    pl.kernel,
    mesh=_mesh,
    out_type=jax.ShapeDtypeStruct((B, H, D), jnp.float32),
    compiler_params=pltpu.CompilerParams(use_tc_tiling_on_sc=False),
    scratch_types=(
        [pltpu.VMEM((SB, H), jnp.int32) for _ in range(NBUF)]
        + [pltpu.VMEM((SB, H, D), jnp.float32) for _ in range(NBUF)]
        + [pltpu.SemaphoreType.DMA for _ in range(3 * NBUF)]
    ),
)
def _gather(idx_hbm, table_hbm, out_hbm, *bufs):
    idx_v = bufs[:NBUF]
    rows_v = bufs[NBUF:2 * NBUF]
    gsem = bufs[2 * NBUF:3 * NBUF]
    osem = bufs[3 * NBUF:4 * NBUF]
    isem = bufs[4 * NBUF:5 * NBUF]

    wid = lax.axis_index("s") * NC + lax.axis_index("c")
    row0 = wid * B_PER_W

    def idx_load(b, c):
        return pltpu.make_async_copy(
            idx_hbm.at[pl.ds(row0 + c * SB, SB)], idx_v[b], isem[b])

    def g_copies(b):
        return [
            pltpu.make_async_copy(
                table_hbm.at[idx_v[b].at[i, pl.ds(h0, hn)]],
                rows_v[b].at[i, pl.ds(h0, hn)], gsem[b])
            for i in range(SB)
            for (h0, hn) in SPLITS
        ]

    def o_copy(b, c):
        return pltpu.make_async_copy(
            rows_v[b], out_hbm.at[pl.ds(row0 + c * SB, SB)], osem[b])

    for m in range(NBUF):
        idx_load(m, m).start()
    for m in range(NBUF - 1):
        idx_load(m, m).wait()
        for cp in g_copies(m):
            cp.start()

    def step(g, carry):
        for b in range(NBUF):
            c = g * NBUF + b
            fb = (b - 1) % NBUF

            if b == 0:
                @pl.when(g > 0)
                def _():
                    o_copy(fb, c - 1).wait()
            else:
                o_copy(fb, c - 1).wait()

            def fire_ahead(c=c, fb=fb):
                idx_load(fb, c + NBUF - 1).wait()
                for cp in g_copies(fb):
                    cp.start()

            if b == 0:
                fire_ahead()
            else:
                pl.when(g < G - 1)(fire_ahead)

            for cp in g_copies(b):
                cp.wait()
            o_copy(b, c).start()

            @pl.when(g < G - 1)
            def _():
                idx_load(b, c + NBUF).start()
        return carry

    lax.fori_loop(0, G, step, 0)
    o_copy((N_CHUNKS - 1) % NBUF, N_CHUNKS - 1).wait()


def kernel(x, emb):
    return _gather(x.astype(jnp.int32), emb)
